# Initial kernel scaffold; baseline (speedup 1.0000x reference)
import jax
import jax.numpy as jnp
from jax.experimental import pallas as pl


def kernel(x, edge_index, W_lin, b_lin, Wl, Wr, att, bias_g, w1, b1, w2, b2, ln_g, ln_b, W_down, b_down, W_out, b_out, W_out2, b_out2):
    raise NotImplementedError("write your pallas kernel here")


# speedup vs baseline: 29.4476x; 29.4476x over previous
import functools
import jax
import jax.numpy as jnp
from jax import lax
from jax.experimental import pallas as pl
from jax.experimental.pallas import tpu as pltpu
from jax.experimental.pallas import tpu_sc as plsc

N = 10000
D = 128
H = 8
C = 16
NP = 10240
NC = 2
NS = 16
NW = NC * NS
CH = 64
SW = 32
DW = D + C
ROWS_PER_SUB = NP // NS
DUMMY = N

f32 = jnp.float32


def _lin_body(x_ref, w_ref, b_ref, o_ref):
    o_ref[...] = jnp.dot(x_ref[...], w_ref[...],
                         preferred_element_type=f32) + b_ref[...]


def _tc_linear(x, w, b):
    blk = 1280
    grid = NP // blk
    return pl.pallas_call(
        _lin_body,
        grid=(grid,),
        in_specs=[
            pl.BlockSpec((blk, D), lambda i: (i, 0)),
            pl.BlockSpec((D, D), lambda i: (0, 0)),
            pl.BlockSpec((1, D), lambda i: (0, 0)),
        ],
        out_specs=pl.BlockSpec((blk, D), lambda i: (i, 0)),
        out_shape=jax.ShapeDtypeStruct((NP, D), f32),
    )(x, w, b)


def _proj_body(h_ref, wl_ref, wr_ref, xl_ref, xr_ref):
    h = h_ref[...]
    xl_ref[...] = jnp.dot(h, wl_ref[...], preferred_element_type=f32)
    xr_ref[...] = jnp.dot(h, wr_ref[...], preferred_element_type=f32)


def _tc_proj(h, wl, wr):
    blk = 1280
    grid = NP // blk
    return pl.pallas_call(
        _proj_body,
        grid=(grid,),
        in_specs=[
            pl.BlockSpec((blk, D), lambda i: (i, 0)),
            pl.BlockSpec((D, D), lambda i: (0, 0)),
            pl.BlockSpec((D, D), lambda i: (0, 0)),
        ],
        out_specs=[
            pl.BlockSpec((blk, D), lambda i: (i, 0)),
            pl.BlockSpec((blk, D), lambda i: (i, 0)),
        ],
        out_shape=[jax.ShapeDtypeStruct((NP, D), f32),
                   jax.ShapeDtypeStruct((NP, D), f32)],
    )(h, wl, wr)


def _ffn_body(do_relu, o2_ref, d2_ref, bg_ref, w1_ref, b1_ref, w2_ref,
              b2_ref, g_ref, bb_ref, out_ref):
    num = o2_ref[0] + o2_ref[1]
    den = (d2_ref[0] + d2_ref[1])[:, :H]
    rcp = 1.0 / (den + 1e-16)
    pieces = [num[:, 16 * h_:16 * (h_ + 1)] * rcp[:, h_:h_ + 1]
              for h_ in range(H)]
    gat = jnp.concatenate(pieces, axis=1) + bg_ref[...]
    y = jnp.maximum(jnp.dot(gat, w1_ref[...], preferred_element_type=f32)
                    + b1_ref[...], 0.0)
    y = jnp.dot(y, w2_ref[...], preferred_element_type=f32) + b2_ref[...]
    z = y + gat
    mu = jnp.mean(z, axis=-1, keepdims=True)
    zc = z - mu
    var = jnp.mean(zc * zc, axis=-1, keepdims=True)
    hn = zc * lax.rsqrt(var + 1e-5) * g_ref[...] + bb_ref[...]
    if do_relu:
        hn = jnp.maximum(hn, 0.0)
    out_ref[...] = hn


def _tc_ffn(o2, d2, bg, w1, b1, w2, b2, g, bb, do_relu):
    blk = 1280
    grid = NP // blk
    return pl.pallas_call(
        functools.partial(_ffn_body, do_relu),
        grid=(grid,),
        in_specs=[
            pl.BlockSpec((2, blk, D), lambda i: (0, i, 0)),
            pl.BlockSpec((2, blk, D), lambda i: (0, i, 0)),
            pl.BlockSpec((1, D), lambda i: (0, 0)),
            pl.BlockSpec((D, D), lambda i: (0, 0)),
            pl.BlockSpec((1, D), lambda i: (0, 0)),
            pl.BlockSpec((D, D), lambda i: (0, 0)),
            pl.BlockSpec((1, D), lambda i: (0, 0)),
            pl.BlockSpec((1, D), lambda i: (0, 0)),
            pl.BlockSpec((1, D), lambda i: (0, 0)),
        ],
        out_specs=pl.BlockSpec((blk, D), lambda i: (i, 0)),
        out_shape=jax.ShapeDtypeStruct((NP, D), f32),
    )(o2, d2, bg, w1, b1, w2, b2, g, bb)


def _final_body(h_ref, wdt_ref, bd_ref, wo_ref, bo_ref, wo2_ref, bo2_ref,
                out_ref, acc_ref):
    k = pl.program_id(0)
    nk = pl.num_programs(0)

    @pl.when(k == 0)
    def _():
        acc_ref[...] = jnp.zeros_like(acc_ref)

    v = jnp.sum(h_ref[...] * wdt_ref[...], axis=1, keepdims=True)
    v = v + bd_ref[...]
    acc_ref[...] += jnp.sum(v * wo_ref[...], axis=0, keepdims=True)

    @pl.when(k == nk - 1)
    def _():
        z = acc_ref[...] + bo_ref[...]
        ql = jnp.maximum(z, 0.01 * z)
        qb = jnp.broadcast_to(ql, (8, ql.shape[1]))
        res = jnp.dot(qb, wo2_ref[...], preferred_element_type=f32)
        out_ref[...] = res[0:1] + bo2_ref[...]


def _tc_final(h, wdt, bd, wo, bo, wo2, bo2):
    hid = wo.shape[1]
    adim = wo2.shape[1]
    blk = 400
    grid = N // blk
    return pl.pallas_call(
        _final_body,
        grid=(grid,),
        in_specs=[
            pl.BlockSpec((blk, D), lambda k: (k, 0)),
            pl.BlockSpec((1, D), lambda k: (0, 0)),
            pl.BlockSpec((1, 1), lambda k: (0, 0)),
            pl.BlockSpec((blk, hid), lambda k: (k, 0)),
            pl.BlockSpec((1, hid), lambda k: (0, 0)),
            pl.BlockSpec((hid, adim), lambda k: (0, 0)),
            pl.BlockSpec((1, adim), lambda k: (0, 0)),
        ],
        out_specs=pl.BlockSpec((1, adim), lambda k: (0, 0)),
        out_shape=jax.ShapeDtypeStruct((1, adim), f32),
        scratch_shapes=[pltpu.VMEM((1, hid), f32)],
    )(h, wdt, bd, wo, bo, wo2, bo2)


_GATHER_DNUMS = lax.GatherDimensionNumbers(
    offset_dims=(), collapsed_slice_dims=(0,), start_index_map=(0,))


def _rot(x, k):
    idx = ((lax.iota(jnp.int32, 16) + k) & 15).reshape(16, 1)
    return lax.gather(x, idx, _GATHER_DNUMS, (1,),
                      mode=lax.GatherScatterMode.PROMISE_IN_BOUNDS)


def _lane_sum(x):
    for k in (8, 4, 2, 1):
        x = x + _rot(x, k)
    return x


def _edge_body(n_chunks, xl_hbm, xr_hbm, src_hbm, dst_hbm, att_hbm,
               out_hbm, ex_hbm,
               src_v, dst_v, xlr, xrr, wv, exv, att_v, out_sh,
               sem1, sem2):
    c = lax.axis_index("c")
    s = lax.axis_index("s")
    wid = c * NS + s
    iota = lax.iota(jnp.int32, 16)
    zero16 = jnp.zeros((16,), f32)

    pltpu.sync_copy(att_hbm, att_v)

    def _zero_buf(i, _):
        for j in range(D // 16):
            wv[i, pl.ds(16 * j, 16)] = zero16
        return 0
    lax.fori_loop(0, CH, _zero_buf, 0)
    for b in range(ROWS_PER_SUB // CH):
        pltpu.sync_copy(wv, out_sh.at[pl.ds(s * ROWS_PER_SUB + b * CH, CH)])
    plsc.subcore_barrier()

    def _chunk(i, _):
        base = wid * (n_chunks * CH) + i * CH

        def _sub_idx(k, _):
            pltpu.sync_copy(src_hbm.at[pl.ds(base + SW * k, SW)],
                            src_v.at[k])
            pltpu.sync_copy(dst_hbm.at[pl.ds(base + SW * k, SW)],
                            dst_v.at[k])
            return 0
        lax.fori_loop(0, CH // SW, _sub_idx, 0)

        def _sub_g(k, _):
            cpa = pltpu.async_copy(xl_hbm.at[src_v.at[k]],
                                   xlr.at[pl.ds(SW * k, SW)], sem1)
            cpb = pltpu.async_copy(xr_hbm.at[dst_v.at[k]],
                                   xrr.at[pl.ds(SW * k, SW)], sem2)
            cpa.wait()
            cpb.wait()
            return 0
        lax.fori_loop(0, CH // SW, _sub_g, 0)

        def _edge(e, _):
            a_list = []
            eh_list = []
            ex_row = zero16
            for h_ in range(H):
                a = xlr[e, pl.ds(16 * h_, 16)]
                b = xrr[e, pl.ds(16 * h_, 16)]
                m = a + b
                m = jnp.maximum(m, 0.2 * m)
                t = m * att_v[h_]
                eh = jnp.exp(_lane_sum(t))
                a_list.append(a)
                eh_list.append(eh)
                ex_row = jnp.where(iota == h_, eh, ex_row)
            for h_ in range(H):
                wv[e, pl.ds(16 * h_, 16)] = eh_list[h_] * a_list[h_]
            exv[e, :] = ex_row
            return 0
        lax.fori_loop(0, CH, _edge, 0)

        pltpu.sync_copy(exv, ex_hbm.at[pl.ds(base, CH)])

        def _sub_s(k, _):
            pltpu.sync_copy(wv.at[pl.ds(SW * k, SW)],
                            out_sh.at[dst_v.at[k]], add=True)
            return 0
        lax.fori_loop(0, CH // SW, _sub_s, 0)
        return 0
    lax.fori_loop(0, n_chunks, _chunk, 0)

    plsc.subcore_barrier()
    row0 = s * ROWS_PER_SUB
    pltpu.sync_copy(out_sh.at[pl.ds(row0, ROWS_PER_SUB)],
                    out_hbm.at[c, pl.ds(row0, ROWS_PER_SUB)])


def _sc_edge(xl, xr, src, dst, att, n_chunks):
    et_pad = src.shape[0]
    mesh = plsc.VectorSubcoreMesh(core_axis_name="c", subcore_axis_name="s",
                                  num_cores=NC, num_subcores=NS)
    fn = pl.kernel(
        functools.partial(_edge_body, n_chunks),
        out_type=[jax.ShapeDtypeStruct((NC, NP, D), f32),
                  jax.ShapeDtypeStruct((et_pad, C), f32)],
        mesh=mesh,
        scratch_types=[
            pltpu.VMEM((CH // SW, SW), jnp.int32),
            pltpu.VMEM((CH // SW, SW), jnp.int32),
            pltpu.VMEM((CH, D), f32),
            pltpu.VMEM((CH, D), f32),
            pltpu.VMEM((CH, D), f32),
            pltpu.VMEM((CH, C), f32),
            pltpu.VMEM((H, C), f32),
            pltpu.VMEM_SHARED((NP, D), f32),
            pltpu.SemaphoreType.DMA,
            pltpu.SemaphoreType.DMA,
        ],
    )
    return fn(xl, xr, src, dst, att)


def _den_body(n_chunks, ex_hbm, dst_hbm, den_hbm, dst_v, exs, wv2, den_sh,
              sem1):
    c = lax.axis_index("c")
    s = lax.axis_index("s")
    wid = c * NS + s
    zero16 = jnp.zeros((16,), f32)

    def _zero_buf(i, _):
        for j in range(D // 16):
            wv2[i, pl.ds(16 * j, 16)] = zero16
        return 0
    lax.fori_loop(0, CH, _zero_buf, 0)
    for b in range(ROWS_PER_SUB // CH):
        pltpu.sync_copy(wv2, den_sh.at[pl.ds(s * ROWS_PER_SUB + b * CH, CH)])
    plsc.subcore_barrier()

    def _chunk(i, _):
        base = wid * (n_chunks * CH) + i * CH

        def _sub_idx(k, _):
            pltpu.sync_copy(dst_hbm.at[pl.ds(base + SW * k, SW)],
                            dst_v.at[k])
            return 0
        lax.fori_loop(0, CH // SW, _sub_idx, 0)
        pltpu.sync_copy(ex_hbm.at[pl.ds(base, CH)], exs)

        def _expand(e, _):
            wv2[e, pl.ds(0, 16)] = exs[e, :]
            return 0
        lax.fori_loop(0, CH, _expand, 0)

        def _sub_s(k, _):
            pltpu.sync_copy(wv2.at[pl.ds(SW * k, SW)],
                            den_sh.at[dst_v.at[k]], add=True)
            return 0
        lax.fori_loop(0, CH // SW, _sub_s, 0)
        return 0
    lax.fori_loop(0, n_chunks, _chunk, 0)

    plsc.subcore_barrier()
    row0 = s * ROWS_PER_SUB
    pltpu.sync_copy(den_sh.at[pl.ds(row0, ROWS_PER_SUB)],
                    den_hbm.at[c, pl.ds(row0, ROWS_PER_SUB)])


def _sc_den(ex, dst, n_chunks):
    mesh = plsc.VectorSubcoreMesh(core_axis_name="c", subcore_axis_name="s",
                                  num_cores=NC, num_subcores=NS)
    fn = pl.kernel(
        functools.partial(_den_body, n_chunks),
        out_type=jax.ShapeDtypeStruct((NC, NP, D), f32),
        mesh=mesh,
        scratch_types=[
            pltpu.VMEM((CH // SW, SW), jnp.int32),
            pltpu.VMEM((CH, C), f32),
            pltpu.VMEM((CH, D), f32),
            pltpu.VMEM_SHARED((NP, D), f32),
            pltpu.SemaphoreType.DMA,
        ],
    )
    return fn(ex, dst)


def kernel(x, edge_index, W_lin, b_lin, Wl, Wr, att, bias_g, w1, b1, w2, b2,
           ln_g, ln_b, W_down, b_down, W_out, b_out, W_out2, b_out2):
    n = x.shape[0]
    e_edges = edge_index.shape[1]
    et = e_edges + n
    n_chunks = -(-et // (NW * CH))
    et_pad = NW * CH * n_chunks

    loops = jnp.arange(n, dtype=edge_index.dtype)
    pad = jnp.full((et_pad - et,), DUMMY, edge_index.dtype)
    src = jnp.concatenate([edge_index[0], loops, pad])
    dst = jnp.concatenate([edge_index[1], loops, pad])

    xp = jnp.zeros((NP, D), f32).at[:n].set(x)
    L = Wl.shape[0]

    h = _tc_linear(xp, W_lin, b_lin.reshape(1, D))
    for i in range(L):
        xl, xr = _tc_proj(h, Wl[i], Wr[i])
        o2, ex = _sc_edge(xl, xr, src, dst, att[i], n_chunks)
        d2 = _sc_den(ex, dst, n_chunks)
        h = _tc_ffn(o2, d2, bias_g[i].reshape(1, D), w1[i],
                    b1[i].reshape(1, D), w2[i], b2[i].reshape(1, D),
                    ln_g[i].reshape(1, D), ln_b[i].reshape(1, D),
                    do_relu=(i < L - 1))

    return _tc_final(h, W_down.reshape(1, D), b_down.reshape(1, 1),
                     W_out, b_out.reshape(1, -1), W_out2,
                     b_out2.reshape(1, -1))


# speedup vs baseline: 36.0019x; 1.2226x over previous
import functools
import jax
import jax.numpy as jnp
from jax import lax
from jax.experimental import pallas as pl
from jax.experimental.pallas import tpu as pltpu
from jax.experimental.pallas import tpu_sc as plsc

N = 10000
D = 128
H = 8
C = 16
NP = 10240
NC = 2
NS = 16
NW = NC * NS
CH = 64
SW = 64
DW = D + C
ROWS_PER_SUB = NP // NS
DUMMY = N

f32 = jnp.float32


def _lin_body(x_ref, w_ref, b_ref, o_ref):
    o_ref[...] = jnp.dot(x_ref[...], w_ref[...],
                         preferred_element_type=f32) + b_ref[...]


def _tc_linear(x, w, b):
    blk = 1280
    grid = NP // blk
    return pl.pallas_call(
        _lin_body,
        grid=(grid,),
        in_specs=[
            pl.BlockSpec((blk, D), lambda i: (i, 0)),
            pl.BlockSpec((D, D), lambda i: (0, 0)),
            pl.BlockSpec((1, D), lambda i: (0, 0)),
        ],
        out_specs=pl.BlockSpec((blk, D), lambda i: (i, 0)),
        out_shape=jax.ShapeDtypeStruct((NP, D), f32),
    )(x, w, b)


def _proj_body(h_ref, wl_ref, wr_ref, xl_ref, xr_ref):
    h = h_ref[...]
    xl_ref[...] = jnp.dot(h, wl_ref[...], preferred_element_type=f32)
    xr_ref[...] = jnp.dot(h, wr_ref[...], preferred_element_type=f32)


def _tc_proj(h, wl, wr):
    blk = 1280
    grid = NP // blk
    return pl.pallas_call(
        _proj_body,
        grid=(grid,),
        in_specs=[
            pl.BlockSpec((blk, D), lambda i: (i, 0)),
            pl.BlockSpec((D, D), lambda i: (0, 0)),
            pl.BlockSpec((D, D), lambda i: (0, 0)),
        ],
        out_specs=[
            pl.BlockSpec((blk, D), lambda i: (i, 0)),
            pl.BlockSpec((blk, D), lambda i: (i, 0)),
        ],
        out_shape=[jax.ShapeDtypeStruct((NP, D), f32),
                   jax.ShapeDtypeStruct((NP, D), f32)],
    )(h, wl, wr)


def _ffn_body(do_relu, o2_ref, d2_ref, bg_ref, w1_ref, b1_ref, w2_ref,
              b2_ref, g_ref, bb_ref, out_ref):
    num = o2_ref[0] + o2_ref[1]
    den = (d2_ref[0] + d2_ref[1])[:, :H]
    rcp = 1.0 / (den + 1e-16)
    pieces = [num[:, 16 * h_:16 * (h_ + 1)] * rcp[:, h_:h_ + 1]
              for h_ in range(H)]
    gat = jnp.concatenate(pieces, axis=1) + bg_ref[...]
    y = jnp.maximum(jnp.dot(gat, w1_ref[...], preferred_element_type=f32)
                    + b1_ref[...], 0.0)
    y = jnp.dot(y, w2_ref[...], preferred_element_type=f32) + b2_ref[...]
    z = y + gat
    mu = jnp.mean(z, axis=-1, keepdims=True)
    zc = z - mu
    var = jnp.mean(zc * zc, axis=-1, keepdims=True)
    hn = zc * lax.rsqrt(var + 1e-5) * g_ref[...] + bb_ref[...]
    if do_relu:
        hn = jnp.maximum(hn, 0.0)
    out_ref[...] = hn


def _tc_ffn(o2, d2, bg, w1, b1, w2, b2, g, bb, do_relu):
    blk = 1280
    grid = NP // blk
    return pl.pallas_call(
        functools.partial(_ffn_body, do_relu),
        grid=(grid,),
        in_specs=[
            pl.BlockSpec((2, blk, D), lambda i: (0, i, 0)),
            pl.BlockSpec((2, blk, D), lambda i: (0, i, 0)),
            pl.BlockSpec((1, D), lambda i: (0, 0)),
            pl.BlockSpec((D, D), lambda i: (0, 0)),
            pl.BlockSpec((1, D), lambda i: (0, 0)),
            pl.BlockSpec((D, D), lambda i: (0, 0)),
            pl.BlockSpec((1, D), lambda i: (0, 0)),
            pl.BlockSpec((1, D), lambda i: (0, 0)),
            pl.BlockSpec((1, D), lambda i: (0, 0)),
        ],
        out_specs=pl.BlockSpec((blk, D), lambda i: (i, 0)),
        out_shape=jax.ShapeDtypeStruct((NP, D), f32),
    )(o2, d2, bg, w1, b1, w2, b2, g, bb)


def _final_body(h_ref, wdt_ref, bd_ref, wo_ref, bo_ref, wo2_ref, bo2_ref,
                out_ref, acc_ref):
    k = pl.program_id(0)
    nk = pl.num_programs(0)

    @pl.when(k == 0)
    def _():
        acc_ref[...] = jnp.zeros_like(acc_ref)

    v = jnp.sum(h_ref[...] * wdt_ref[...], axis=1, keepdims=True)
    v = v + bd_ref[...]
    acc_ref[...] += jnp.sum(v * wo_ref[...], axis=0, keepdims=True)

    @pl.when(k == nk - 1)
    def _():
        z = acc_ref[...] + bo_ref[...]
        ql = jnp.maximum(z, 0.01 * z)
        qb = jnp.broadcast_to(ql, (8, ql.shape[1]))
        res = jnp.dot(qb, wo2_ref[...], preferred_element_type=f32)
        out_ref[...] = res[0:1] + bo2_ref[...]


def _tc_final(h, wdt, bd, wo, bo, wo2, bo2):
    hid = wo.shape[1]
    adim = wo2.shape[1]
    blk = 400
    grid = N // blk
    return pl.pallas_call(
        _final_body,
        grid=(grid,),
        in_specs=[
            pl.BlockSpec((blk, D), lambda k: (k, 0)),
            pl.BlockSpec((1, D), lambda k: (0, 0)),
            pl.BlockSpec((1, 1), lambda k: (0, 0)),
            pl.BlockSpec((blk, hid), lambda k: (k, 0)),
            pl.BlockSpec((1, hid), lambda k: (0, 0)),
            pl.BlockSpec((hid, adim), lambda k: (0, 0)),
            pl.BlockSpec((1, adim), lambda k: (0, 0)),
        ],
        out_specs=pl.BlockSpec((1, adim), lambda k: (0, 0)),
        out_shape=jax.ShapeDtypeStruct((1, adim), f32),
        scratch_shapes=[pltpu.VMEM((1, hid), f32)],
    )(h, wdt, bd, wo, bo, wo2, bo2)


_GATHER_DNUMS = lax.GatherDimensionNumbers(
    offset_dims=(), collapsed_slice_dims=(0,), start_index_map=(0,))


def _rot(x, k):
    idx = ((lax.iota(jnp.int32, 16) + k) & 15).reshape(16, 1)
    return lax.gather(x, idx, _GATHER_DNUMS, (1,),
                      mode=lax.GatherScatterMode.PROMISE_IN_BOUNDS)


def _lane_sum(x):
    for k in (8, 4, 2, 1):
        x = x + _rot(x, k)
    return x


def _edge_body(n_chunks, xl_hbm, xr_hbm, src_hbm, dst_hbm, att_hbm,
               out_hbm, ex_hbm,
               src_v, dst_v, xlr, xrr, wv, exv, att_v, out_sh,
               sem1, sem2):
    c = lax.axis_index("c")
    s = lax.axis_index("s")
    wid = c * NS + s
    iota = lax.iota(jnp.int32, 16)
    zero16 = jnp.zeros((16,), f32)

    pltpu.sync_copy(att_hbm, att_v)

    def _zero_buf(i, _):
        for j in range(D // 16):
            wv[i, pl.ds(16 * j, 16)] = zero16
        return 0
    lax.fori_loop(0, CH, _zero_buf, 0)
    for b in range(ROWS_PER_SUB // CH):
        pltpu.sync_copy(wv, out_sh.at[pl.ds(s * ROWS_PER_SUB + b * CH, CH)])
    plsc.subcore_barrier()

    def _chunk(i, _):
        base = wid * (n_chunks * CH) + i * CH

        def _sub_idx(k, _):
            pltpu.sync_copy(src_hbm.at[pl.ds(base + SW * k, SW)],
                            src_v.at[k])
            pltpu.sync_copy(dst_hbm.at[pl.ds(base + SW * k, SW)],
                            dst_v.at[k])
            return 0
        lax.fori_loop(0, CH // SW, _sub_idx, 0)

        def _sub_g(k, _):
            cpa = pltpu.async_copy(xl_hbm.at[src_v.at[k]],
                                   xlr.at[pl.ds(SW * k, SW)], sem1)
            cpb = pltpu.async_copy(xr_hbm.at[dst_v.at[k]],
                                   xrr.at[pl.ds(SW * k, SW)], sem2)
            cpa.wait()
            cpb.wait()
            return 0
        lax.fori_loop(0, CH // SW, _sub_g, 0)

        def _edge(e, _):
            a_list = []
            eh_list = []
            ex_row = zero16
            for h_ in range(H):
                a = xlr[e, pl.ds(16 * h_, 16)]
                b = xrr[e, pl.ds(16 * h_, 16)]
                m = a + b
                m = jnp.maximum(m, 0.2 * m)
                t = m * att_v[h_]
                eh = jnp.exp(_lane_sum(t))
                a_list.append(a)
                eh_list.append(eh)
                ex_row = jnp.where(iota == h_, eh, ex_row)
            for h_ in range(H):
                wv[e, pl.ds(16 * h_, 16)] = eh_list[h_] * a_list[h_]
            exv[e, :] = ex_row
            return 0
        lax.fori_loop(0, CH, _edge, 0)

        pltpu.sync_copy(exv, ex_hbm.at[pl.ds(base, CH)])

        def _sub_s(k, _):
            pltpu.sync_copy(wv.at[pl.ds(SW * k, SW)],
                            out_sh.at[dst_v.at[k]], add=True)
            return 0
        lax.fori_loop(0, CH // SW, _sub_s, 0)
        return 0
    lax.fori_loop(0, n_chunks, _chunk, 0)

    plsc.subcore_barrier()
    row0 = s * ROWS_PER_SUB
    pltpu.sync_copy(out_sh.at[pl.ds(row0, ROWS_PER_SUB)],
                    out_hbm.at[c, pl.ds(row0, ROWS_PER_SUB)])


def _sc_edge(xl, xr, src, dst, att, n_chunks):
    et_pad = src.shape[0]
    mesh = plsc.VectorSubcoreMesh(core_axis_name="c", subcore_axis_name="s",
                                  num_cores=NC, num_subcores=NS)
    fn = pl.kernel(
        functools.partial(_edge_body, n_chunks),
        out_type=[jax.ShapeDtypeStruct((NC, NP, D), f32),
                  jax.ShapeDtypeStruct((et_pad, C), f32)],
        mesh=mesh,
        scratch_types=[
            pltpu.VMEM((CH // SW, SW), jnp.int32),
            pltpu.VMEM((CH // SW, SW), jnp.int32),
            pltpu.VMEM((CH, D), f32),
            pltpu.VMEM((CH, D), f32),
            pltpu.VMEM((CH, D), f32),
            pltpu.VMEM((CH, C), f32),
            pltpu.VMEM((H, C), f32),
            pltpu.VMEM_SHARED((NP, D), f32),
            pltpu.SemaphoreType.DMA,
            pltpu.SemaphoreType.DMA,
        ],
    )
    return fn(xl, xr, src, dst, att)


def _den_body(n_chunks, ex_hbm, dst_hbm, den_hbm, dst_v, exs, wv2, den_sh,
              sem1):
    c = lax.axis_index("c")
    s = lax.axis_index("s")
    wid = c * NS + s
    zero16 = jnp.zeros((16,), f32)

    def _zero_buf(i, _):
        for j in range(D // 16):
            wv2[i, pl.ds(16 * j, 16)] = zero16
        return 0
    lax.fori_loop(0, CH, _zero_buf, 0)
    for b in range(ROWS_PER_SUB // CH):
        pltpu.sync_copy(wv2, den_sh.at[pl.ds(s * ROWS_PER_SUB + b * CH, CH)])
    plsc.subcore_barrier()

    def _chunk(i, _):
        base = wid * (n_chunks * CH) + i * CH

        def _sub_idx(k, _):
            pltpu.sync_copy(dst_hbm.at[pl.ds(base + SW * k, SW)],
                            dst_v.at[k])
            return 0
        lax.fori_loop(0, CH // SW, _sub_idx, 0)
        pltpu.sync_copy(ex_hbm.at[pl.ds(base, CH)], exs)

        def _expand(e, _):
            wv2[e, pl.ds(0, 16)] = exs[e, :]
            return 0
        lax.fori_loop(0, CH, _expand, 0)

        def _sub_s(k, _):
            pltpu.sync_copy(wv2.at[pl.ds(SW * k, SW)],
                            den_sh.at[dst_v.at[k]], add=True)
            return 0
        lax.fori_loop(0, CH // SW, _sub_s, 0)
        return 0
    lax.fori_loop(0, n_chunks, _chunk, 0)

    plsc.subcore_barrier()
    row0 = s * ROWS_PER_SUB
    pltpu.sync_copy(den_sh.at[pl.ds(row0, ROWS_PER_SUB)],
                    den_hbm.at[c, pl.ds(row0, ROWS_PER_SUB)])


def _sc_den(ex, dst, n_chunks):
    mesh = plsc.VectorSubcoreMesh(core_axis_name="c", subcore_axis_name="s",
                                  num_cores=NC, num_subcores=NS)
    fn = pl.kernel(
        functools.partial(_den_body, n_chunks),
        out_type=jax.ShapeDtypeStruct((NC, NP, D), f32),
        mesh=mesh,
        scratch_types=[
            pltpu.VMEM((CH // SW, SW), jnp.int32),
            pltpu.VMEM((CH, C), f32),
            pltpu.VMEM((CH, D), f32),
            pltpu.VMEM_SHARED((NP, D), f32),
            pltpu.SemaphoreType.DMA,
        ],
    )
    return fn(ex, dst)


def kernel(x, edge_index, W_lin, b_lin, Wl, Wr, att, bias_g, w1, b1, w2, b2,
           ln_g, ln_b, W_down, b_down, W_out, b_out, W_out2, b_out2):
    n = x.shape[0]
    e_edges = edge_index.shape[1]
    et = e_edges + n
    n_chunks = -(-et // (NW * CH))
    et_pad = NW * CH * n_chunks

    loops = jnp.arange(n, dtype=edge_index.dtype)
    pad = jnp.full((et_pad - et,), DUMMY, edge_index.dtype)
    src = jnp.concatenate([edge_index[0], loops, pad])
    dst = jnp.concatenate([edge_index[1], loops, pad])

    xp = jnp.zeros((NP, D), f32).at[:n].set(x)
    L = Wl.shape[0]

    h = _tc_linear(xp, W_lin, b_lin.reshape(1, D))
    for i in range(L):
        xl, xr = _tc_proj(h, Wl[i], Wr[i])
        o2, ex = _sc_edge(xl, xr, src, dst, att[i], n_chunks)
        d2 = _sc_den(ex, dst, n_chunks)
        h = _tc_ffn(o2, d2, bias_g[i].reshape(1, D), w1[i],
                    b1[i].reshape(1, D), w2[i], b2[i].reshape(1, D),
                    ln_g[i].reshape(1, D), ln_b[i].reshape(1, D),
                    do_relu=(i < L - 1))

    return _tc_final(h, W_down.reshape(1, D), b_down.reshape(1, 1),
                     W_out, b_out.reshape(1, -1), W_out2,
                     b_out2.reshape(1, -1))
